# baseline (speedup 1.0000x reference)
import functools

import jax
import jax.numpy as jnp
from jax import lax
from jax.experimental import pallas as pl
from jax.experimental.pallas import tpu as pltpu
from jax.experimental.pallas import tpu_sc as plsc

NUM_CORES = 2
NUM_SUBCORES = 16
NW = NUM_CORES * NUM_SUBCORES
LANES = 16
B = 16384
D = 32
TC_B = 4096
SC_B = B - TC_B
EPW = SC_B // NW
GRP = 2
NBUF = 4
NK = EPW // LANES
TC_G = 8
TC_STEPS = TC_B // TC_G

_mesh = plsc.VectorSubcoreMesh(core_axis_name="c", subcore_axis_name="s")


@functools.partial(
    pl.kernel,
    mesh=_mesh,
    compiler_params=pltpu.CompilerParams(
        needs_layout_passes=False, use_tc_tiling_on_sc=True),
    out_type=jax.ShapeDtypeStruct((SC_B,), jnp.float32),
    scratch_types=[
        pltpu.VMEM((EPW,), jnp.int32),
        pltpu.VMEM((EPW,), jnp.int32),
        pltpu.VMEM((NBUF, GRP, D, 128), jnp.float32),
        pltpu.VMEM((NBUF, GRP, D, 128), jnp.float32),
        pltpu.VMEM((LANES,), jnp.float32),
        pltpu.VMEM((EPW,), jnp.float32),
        pltpu.SemaphoreType.DMA,
        pltpu.SemaphoreType.DMA,
    ],
)
def _svd_score_sc(uid_hbm, iid_hbm, ut_hbm, it_hbm, bias_hbm, out_hbm,
                  uidv, iidv, ublk, iblk, biasv, outv, semu, semi):
    wid = lax.axis_index("s") * NUM_CORES + lax.axis_index("c")
    base = wid * EPW

    pltpu.sync_copy(uid_hbm.at[wid], uidv)
    pltpu.sync_copy(iid_hbm.at[wid], iidv)
    pltpu.sync_copy(bias_hbm, biasv)

    lane = lax.iota(jnp.int32, LANES)
    slotv = lane & (GRP - 1)
    pmask = [((lane >> 1) == p).astype(jnp.float32) for p in range(8)]
    bias = biasv[...]

    def fire(uvec, ivec, lane_off, buf):
        for l in range(GRP):
            ub = pl.multiple_of((uvec[lane_off + l] >> 7) << 7, 128)
            pltpu.async_copy(ut_hbm.at[:, pl.ds(ub, 128)], ublk.at[buf, l],
                             semu)
            ib = pl.multiple_of((ivec[lane_off + l] >> 7) << 7, 128)
            pltpu.async_copy(it_hbm.at[:, pl.ds(ib, 128)], iblk.at[buf, l],
                             semi)

    def drain(buf):
        for l in range(GRP):
            pltpu.make_async_copy(
                ut_hbm.at[:, pl.ds(0, 128)], ublk.at[buf, l], semu).wait()
            pltpu.make_async_copy(
                it_hbm.at[:, pl.ds(0, 128)], iblk.at[buf, l], semi).wait()

    def comp(uvec, ivec, buf):
        bufv = jnp.full((LANES,), buf, jnp.int32)
        ulane = uvec & 127
        ilane = ivec & 127
        acc = jnp.zeros((LANES,), jnp.float32)
        for d in range(D):
            dvec = jnp.full((LANES,), d, jnp.int32)
            uu = plsc.load_gather(ublk, [bufv, slotv, dvec, ulane])
            ii = plsc.load_gather(iblk, [bufv, slotv, dvec, ilane])
            acc = acc + uu * ii
        return acc

    uvec0 = uidv[pl.ds(0, LANES)]
    ivec0 = iidv[pl.ds(0, LANES)]
    for p in range(3):
        fire(uvec0, ivec0, p * GRP, p)

    def body(k, carry):
        uvec = uidv[pl.ds(k * LANES, LANES)]
        ivec = iidv[pl.ds(k * LANES, LANES)]
        nvec_u = uidv[pl.ds(jnp.minimum(k + 1, NK - 1) * LANES, LANES)]
        nvec_i = iidv[pl.ds(jnp.minimum(k + 1, NK - 1) * LANES, LANES)]
        accs = []
        for p in range(8):
            nxt = p + 3
            if nxt < 8:
                fire(uvec, ivec, nxt * GRP, nxt & 3)
            else:
                @pl.when(k < NK - 1)
                def _fire_next(nxt=nxt):
                    fire(nvec_u, nvec_i, (nxt - 8) * GRP, nxt & 3)

            drain(p & 3)
            accs.append(comp(uvec, ivec, p & 3))
        res = bias
        for p in range(8):
            res = res + accs[p] * pmask[p]
        outv[pl.ds(k * LANES, LANES)] = res
        return carry

    lax.fori_loop(0, NK, body, 0)

    pltpu.sync_copy(outv, out_hbm.at[pl.ds(base, EPW)])


def _tc_body(uid_sref, iid_sref, *refs):
    ublks = refs[:TC_G]
    iblks = refs[TC_G:2 * TC_G]
    out_ref = refs[2 * TC_G]
    i = pl.program_id(0)
    lanecol = lax.broadcasted_iota(jnp.int32, (D, 128), 1)
    rows = lax.broadcasted_iota(jnp.int32, (TC_G, 128), 0)
    res = jnp.zeros((TC_G, 128), jnp.float32)
    for s in range(TC_G):
        cu = uid_sref[i * TC_G + s] & 127
        ci = iid_sref[i * TC_G + s] & 127
        uval = jnp.sum(
            jnp.where(lanecol == cu, ublks[s][...], 0.0), axis=1,
            keepdims=True)
        ival = jnp.sum(
            jnp.where(lanecol == ci, iblks[s][...], 0.0), axis=1,
            keepdims=True)
        score = jnp.sum(uval * ival)
        res = res + jnp.where(rows == s, score, 0.0)
    out_ref[...] = res[None]


def _tc_gather_dot(uids, iids, ut_t, it_t):
    u_specs = [
        pl.BlockSpec((D, 128),
                     (lambda i, u, it, s=s: (0, u[i * TC_G + s] // 128)))
        for s in range(TC_G)
    ]
    i_specs = [
        pl.BlockSpec((D, 128),
                     (lambda i, u, it, s=s: (0, it[i * TC_G + s] // 128)))
        for s in range(TC_G)
    ]
    out = pl.pallas_call(
        _tc_body,
        grid_spec=pltpu.PrefetchScalarGridSpec(
            num_scalar_prefetch=2,
            grid=(TC_STEPS,),
            in_specs=u_specs + i_specs,
            out_specs=pl.BlockSpec((1, TC_G, 128), lambda i, u, it: (i, 0, 0)),
        ),
        out_shape=jax.ShapeDtypeStruct((TC_STEPS, TC_G, 128), jnp.float32),
    )(uids, iids, *([ut_t] * TC_G), *([it_t] * TC_G))
    return out[:, :, 0].reshape(TC_B)


def kernel(user_ids, item_ids, user_table, item_table, user_bias, item_bias):
    uid = user_ids.astype(jnp.int32)
    iid = item_ids.astype(jnp.int32)
    ut_t = user_table.T
    it_t = item_table.T
    bias16 = jnp.broadcast_to(
        (3.5 + user_bias + item_bias).astype(jnp.float32), (LANES,))
    sc = _svd_score_sc(uid[:SC_B].reshape(NW, EPW),
                       iid[:SC_B].reshape(NW, EPW),
                       ut_t, it_t, bias16)
    tc = _tc_gather_dot(uid[SC_B:], iid[SC_B:], ut_t, it_t)
    tc = tc + (3.5 + user_bias + item_bias).astype(jnp.float32)[0]
    return jnp.concatenate([sc, tc]).reshape(B, 1)


# speedup vs baseline: 2.1218x; 2.1218x over previous
import functools

import jax
import jax.numpy as jnp
from jax import lax
from jax.experimental import pallas as pl
from jax.experimental.pallas import tpu as pltpu
from jax.experimental.pallas import tpu_sc as plsc

NUM_CORES = 2
NUM_SUBCORES = 16
NW = NUM_CORES * NUM_SUBCORES
LANES = 16
B = 16384
D = 32
EPW = B // NW
GRP = 2
NBUF = 4
NK = EPW // LANES

_mesh = plsc.VectorSubcoreMesh(core_axis_name="c", subcore_axis_name="s")


@functools.partial(
    pl.kernel,
    mesh=_mesh,
    compiler_params=pltpu.CompilerParams(
        needs_layout_passes=False, use_tc_tiling_on_sc=True),
    out_type=jax.ShapeDtypeStruct((B,), jnp.float32),
    scratch_types=[
        pltpu.VMEM((EPW,), jnp.int32),
        pltpu.VMEM((EPW,), jnp.int32),
        pltpu.VMEM((NBUF, GRP, D, 128), jnp.float32),
        pltpu.VMEM((NBUF, GRP, D, 128), jnp.float32),
        pltpu.VMEM((LANES,), jnp.float32),
        pltpu.VMEM((EPW,), jnp.float32),
        pltpu.SemaphoreType.DMA,
        pltpu.SemaphoreType.DMA,
    ],
)
def _svd_score(uid_hbm, iid_hbm, ut_hbm, it_hbm, bias_hbm, out_hbm,
               uidv, iidv, ublk, iblk, biasv, outv, semu, semi):
    wid = lax.axis_index("s") * NUM_CORES + lax.axis_index("c")
    base = wid * EPW

    pltpu.sync_copy(uid_hbm.at[wid], uidv)
    pltpu.sync_copy(iid_hbm.at[wid], iidv)
    pltpu.sync_copy(bias_hbm, biasv)

    lane = lax.iota(jnp.int32, LANES)
    slotv = lane & (GRP - 1)
    pmask = [((lane >> 1) == p).astype(jnp.float32) for p in range(8)]
    bias = biasv[...]

    def fire(uvec, ivec, lane_off, buf):
        for l in range(GRP):
            ub = pl.multiple_of((uvec[lane_off + l] >> 7) << 7, 128)
            pltpu.async_copy(ut_hbm.at[:, pl.ds(ub, 128)], ublk.at[buf, l],
                             semu)
            ib = pl.multiple_of((ivec[lane_off + l] >> 7) << 7, 128)
            pltpu.async_copy(it_hbm.at[:, pl.ds(ib, 128)], iblk.at[buf, l],
                             semi)

    def drain(buf):
        for l in range(GRP):
            pltpu.make_async_copy(
                ut_hbm.at[:, pl.ds(0, 128)], ublk.at[buf, l], semu).wait()
            pltpu.make_async_copy(
                it_hbm.at[:, pl.ds(0, 128)], iblk.at[buf, l], semi).wait()

    def comp(uvec, ivec, buf):
        bufv = jnp.full((LANES,), buf, jnp.int32)
        ulane = uvec & 127
        ilane = ivec & 127
        acc = jnp.zeros((LANES,), jnp.float32)
        for d in range(D):
            dvec = jnp.full((LANES,), d, jnp.int32)
            uu = plsc.load_gather(ublk, [bufv, slotv, dvec, ulane])
            ii = plsc.load_gather(iblk, [bufv, slotv, dvec, ilane])
            acc = acc + uu * ii
        return acc

    uvec0 = uidv[pl.ds(0, LANES)]
    ivec0 = iidv[pl.ds(0, LANES)]
    for p in range(3):
        fire(uvec0, ivec0, p * GRP, p)

    def body(k, carry):
        uvec = uidv[pl.ds(k * LANES, LANES)]
        ivec = iidv[pl.ds(k * LANES, LANES)]
        nvec_u = uidv[pl.ds(jnp.minimum(k + 1, NK - 1) * LANES, LANES)]
        nvec_i = iidv[pl.ds(jnp.minimum(k + 1, NK - 1) * LANES, LANES)]
        accs = []
        for p in range(8):
            nxt = p + 3
            if nxt < 8:
                fire(uvec, ivec, nxt * GRP, nxt & 3)
            else:
                @pl.when(k < NK - 1)
                def _fire_next(nxt=nxt):
                    fire(nvec_u, nvec_i, (nxt - 8) * GRP, nxt & 3)

            drain(p & 3)
            accs.append(comp(uvec, ivec, p & 3))
        res = bias
        for p in range(8):
            res = res + accs[p] * pmask[p]
        outv[pl.ds(k * LANES, LANES)] = res
        return carry

    lax.fori_loop(0, NK, body, 0)

    pltpu.sync_copy(outv, out_hbm.at[pl.ds(base, EPW)])


def kernel(user_ids, item_ids, user_table, item_table, user_bias, item_bias):
    uid = user_ids.astype(jnp.int32).reshape(NW, EPW)
    iid = item_ids.astype(jnp.int32).reshape(NW, EPW)
    bias16 = jnp.broadcast_to(
        (3.5 + user_bias + item_bias).astype(jnp.float32), (LANES,))
    score = _svd_score(uid, iid, user_table.T, item_table.T, bias16)
    return score.reshape(B, 1)
